# baseline (device time: 66107 ns/iter reference)
import jax
import jax.numpy as jnp
from jax import lax
from jax.experimental import pallas as pl
from jax.experimental.pallas import tpu as pltpu

NY = 4
NZ = 4
STEPS = 3
RX = 1024
S = 192
H = 96


def kernel(x):
    m, n = x.shape
    MESH = pl.DeviceIdType.MESH

    def body(
        x_ref, out_ref, sraw, rraw, sbulk, rbulk, abuf, bbuf,
        xsem_s, xsem_r, bsem_s, bsem_r, ssems, rsems,
    ):
        my_x = lax.axis_index("x")
        my_y = lax.axis_index("y")
        my_z = lax.axis_index("z")
        idx = my_y * NZ + my_z

        bar = pltpu.get_barrier_semaphore()

        def sig(dev):
            pl.semaphore_signal(bar, inc=1, device_id=dev, device_id_type=MESH)

        ym = jnp.maximum(my_y - 1, 0)
        yp = jnp.minimum(my_y + 1, NY - 1)
        zm = jnp.maximum(my_z - 1, 0)
        zp = jnp.minimum(my_z + 1, NZ - 1)

        sig((1 - my_x, my_y, my_z))

        @pl.when(my_y > 0)
        def _():
            sig((my_x, ym, my_z))

        @pl.when(my_y < NY - 1)
        def _():
            sig((my_x, yp, my_z))

        @pl.when(my_z > 0)
        def _():
            sig((my_x, my_y, zm))

        @pl.when(my_z < NZ - 1)
        def _():
            sig((my_x, my_y, zp))

        pl.semaphore_wait(bar, 1)

        @pl.when(my_y > 0)
        def _():
            pl.semaphore_wait(bar, 1)

        @pl.when(my_y < NY - 1)
        def _():
            pl.semaphore_wait(bar, 1)

        @pl.when(my_z > 0)
        def _():
            pl.semaphore_wait(bar, 1)

        @pl.when(my_z < NZ - 1)
        def _():
            pl.semaphore_wait(bar, 1)

        xpeer = (1 - my_x, my_y, my_z)
        row0 = RX + idx * S
        xs = x_ref[pl.ds(row0, S), :]
        sraw[...] = xs.astype(jnp.bfloat16)
        rx = pltpu.make_async_remote_copy(
            src_ref=sraw,
            dst_ref=rraw,
            send_sem=xsem_s,
            recv_sem=xsem_r,
            device_id=xpeer,
            device_id_type=MESH,
        )
        rx.start()
        sbulk[...] = x_ref[0:RX, :].astype(jnp.bfloat16)
        rbx = pltpu.make_async_remote_copy(
            src_ref=sbulk,
            dst_ref=rbulk,
            send_sem=bsem_s,
            recv_sem=bsem_r,
            device_id=xpeer,
            device_id_type=MESH,
        )
        rbx.start()
        rx.wait()
        ssum = (xs + rraw[...].astype(jnp.float32)).astype(jnp.bfloat16)
        abuf[pl.ds(my_z, 1), pl.ds(my_y, 1), :, :] = ssum[None, None, 0:H, :]
        bbuf[pl.ds(my_y, 1), pl.ds(my_z, 1), :, :] = ssum[None, None, H:S, :]

        def reg_p1A(c):
            return abuf.at[pl.ds(my_z, 1), pl.ds(c, 1)]

        def reg_p1B(c):
            return bbuf.at[pl.ds(my_y, 1), pl.ds(c, 1)]

        def reg_p2A(c):
            return abuf.at[pl.ds(c, 1)]

        def reg_p2B(c):
            return bbuf.at[pl.ds(c, 1)]

        def dev(axis, d):
            if axis == "y":
                return (my_x, jnp.clip(my_y + d, 0, NY - 1), my_z)
            return (my_x, my_y, jnp.clip(my_z + d, 0, NZ - 1))

        phase1 = [
            (my_y, "y", +1, reg_p1A),
            (my_y, "y", -1, reg_p1A),
            (my_z, "z", +1, reg_p1B),
            (my_z, "z", -1, reg_p1B),
        ]
        phase2 = [
            (my_z, "z", +1, reg_p2A),
            (my_z, "z", -1, reg_p2A),
            (my_y, "y", +1, reg_p2B),
            (my_y, "y", -1, reg_p2B),
        ]

        def send_cond_chunk(pos, d, s):
            if d == +1:
                return (pos < 3) & (pos - s >= 0), jnp.clip(pos - s, 0, 3)
            return (pos > 0) & (pos + s <= 3), jnp.clip(pos + s, 0, 3)

        def recv_cond_chunk(pos, d, s):
            if d == +1:
                return (pos > 0) & (pos - 1 - s >= 0), jnp.clip(pos - 1 - s, 0, 3)
            return (pos < 3) & (pos + 1 + s <= 3), jnp.clip(pos + 1 + s, 0, 3)

        def mk(reg, cc, fidx, s, axis, d):
            return pltpu.make_async_remote_copy(
                src_ref=reg(cc),
                dst_ref=reg(cc),
                send_sem=ssems.at[fidx, s],
                recv_sem=rsems.at[fidx, s],
                device_id=dev(axis, d),
                device_id_type=MESH,
            )

        def run_phase(flows, base, after_step=None):
            for s in range(STEPS):
                for fi, (pos, axis, d, reg) in enumerate(flows):
                    cond, cc = send_cond_chunk(pos, d, s)

                    @pl.when(cond)
                    def _(reg=reg, cc=cc, fidx=base + fi, s=s, axis=axis, d=d):
                        mk(reg, cc, fidx, s, axis, d).start()

                for fi, (pos, axis, d, reg) in enumerate(flows):
                    cond, cc = recv_cond_chunk(pos, d, s)

                    @pl.when(cond)
                    def _(reg=reg, cc=cc, fidx=base + fi, s=s, axis=axis, d=d):
                        mk(reg, cc, fidx, s, axis, -d).wait_recv()

                if after_step and s in after_step:
                    after_step[s]()

            for s in range(STEPS):
                for fi, (pos, axis, d, reg) in enumerate(flows):
                    cond, cc = send_cond_chunk(pos, d, s)

                    @pl.when(cond)
                    def _(reg=reg, cc=cc, fidx=base + fi, s=s, axis=axis, d=d):
                        mk(reg, cc, fidx, s, axis, d).wait_send()

        def bulk_add():
            rbx.wait()
            out_ref[0:RX, :] = x_ref[0:RX, :] + rbulk[...].astype(jnp.float32)

        run_phase(phase1, 0)
        run_phase(phase2, 4, after_step={0: bulk_add})

        for yy in range(NY):
            for zz in range(NZ):
                r0 = RX + (yy * NZ + zz) * S
                out_ref[r0:r0 + H, :] = abuf[zz, yy].astype(jnp.float32)
                out_ref[r0 + H:r0 + S, :] = bbuf[yy, zz].astype(jnp.float32)

    return pl.pallas_call(
        body,
        out_shape=jax.ShapeDtypeStruct((m, n), jnp.float32),
        in_specs=[pl.BlockSpec(memory_space=pltpu.VMEM)],
        out_specs=pl.BlockSpec(memory_space=pltpu.VMEM),
        scratch_shapes=[
            pltpu.VMEM((S, n), jnp.bfloat16),
            pltpu.VMEM((S, n), jnp.bfloat16),
            pltpu.VMEM((RX, n), jnp.bfloat16),
            pltpu.VMEM((RX, n), jnp.bfloat16),
            pltpu.VMEM((NZ, NY, H, n), jnp.bfloat16),
            pltpu.VMEM((NY, NZ, H, n), jnp.bfloat16),
            pltpu.SemaphoreType.DMA,
            pltpu.SemaphoreType.DMA,
            pltpu.SemaphoreType.DMA,
            pltpu.SemaphoreType.DMA,
            pltpu.SemaphoreType.DMA((8, STEPS)),
            pltpu.SemaphoreType.DMA((8, STEPS)),
        ],
        compiler_params=pltpu.CompilerParams(collective_id=0),
    )(x)


# device time: 59685 ns/iter; 1.1076x vs baseline; 1.1076x over previous
import jax
import jax.numpy as jnp
from jax import lax
from jax.experimental import pallas as pl
from jax.experimental.pallas import tpu as pltpu

NY = 4
NZ = 4
STEPS = 3
RX = 1536
RXH = 768
S = 160
H = 80


def kernel(x):
    m, n = x.shape
    MESH = pl.DeviceIdType.MESH
    f32 = jnp.float32

    def body(
        x_ref, out_ref, sraw, rraw, sbulk, rbulk, abuf, bbuf,
        xsem_s, xsem_r, bsems_s, bsems_r, ssems, rsems,
    ):
        my_x = lax.axis_index("x")
        my_y = lax.axis_index("y")
        my_z = lax.axis_index("z")
        idx = my_y * NZ + my_z

        bar = pltpu.get_barrier_semaphore()

        def sig(dev):
            pl.semaphore_signal(bar, inc=1, device_id=dev, device_id_type=MESH)

        ym = jnp.maximum(my_y - 1, 0)
        yp = jnp.minimum(my_y + 1, NY - 1)
        zm = jnp.maximum(my_z - 1, 0)
        zp = jnp.minimum(my_z + 1, NZ - 1)

        sig((1 - my_x, my_y, my_z))

        @pl.when(my_y > 0)
        def _():
            sig((my_x, ym, my_z))

        @pl.when(my_y < NY - 1)
        def _():
            sig((my_x, yp, my_z))

        @pl.when(my_z > 0)
        def _():
            sig((my_x, my_y, zm))

        @pl.when(my_z < NZ - 1)
        def _():
            sig((my_x, my_y, zp))

        pl.semaphore_wait(bar, 1)

        @pl.when(my_y > 0)
        def _():
            pl.semaphore_wait(bar, 1)

        @pl.when(my_y < NY - 1)
        def _():
            pl.semaphore_wait(bar, 1)

        @pl.when(my_z > 0)
        def _():
            pl.semaphore_wait(bar, 1)

        @pl.when(my_z < NZ - 1)
        def _():
            pl.semaphore_wait(bar, 1)

        xpeer = (1 - my_x, my_y, my_z)
        row0 = RX + idx * S
        xs = x_ref[pl.ds(row0, S), :]
        sraw[...] = xs.astype(jnp.bfloat16)
        rx = pltpu.make_async_remote_copy(
            src_ref=sraw,
            dst_ref=rraw,
            send_sem=xsem_s,
            recv_sem=xsem_r,
            device_id=xpeer,
            device_id_type=MESH,
        )
        rx.start()
        sbulk[...] = x_ref[0:RX, :].astype(jnp.bfloat16)
        rbx1 = pltpu.make_async_remote_copy(
            src_ref=sbulk.at[0:RXH],
            dst_ref=rbulk.at[0:RXH],
            send_sem=bsems_s.at[0],
            recv_sem=bsems_r.at[0],
            device_id=xpeer,
            device_id_type=MESH,
        )
        rbx1.start()
        rbx2 = pltpu.make_async_remote_copy(
            src_ref=sbulk.at[RXH:RX],
            dst_ref=rbulk.at[RXH:RX],
            send_sem=bsems_s.at[1],
            recv_sem=bsems_r.at[1],
            device_id=xpeer,
            device_id_type=MESH,
        )
        rbx2.start()
        rx.wait()
        ssum = (xs + rraw[...].astype(f32)).astype(jnp.bfloat16)
        abuf[pl.ds(my_z, 1), pl.ds(my_y, 1), :, :] = ssum[None, None, 0:H, :]
        bbuf[pl.ds(my_y, 1), pl.ds(my_z, 1), :, :] = ssum[None, None, H:S, :]

        def reg_p1A(c):
            return abuf.at[pl.ds(my_z, 1), pl.ds(c, 1)]

        def reg_p1B(c):
            return bbuf.at[pl.ds(my_y, 1), pl.ds(c, 1)]

        def dev(axis, d):
            if axis == "y":
                return (my_x, jnp.clip(my_y + d, 0, NY - 1), my_z)
            return (my_x, my_y, jnp.clip(my_z + d, 0, NZ - 1))

        phase1 = [
            (my_y, "y", +1, reg_p1A),
            (my_y, "y", -1, reg_p1A),
            (my_z, "z", +1, reg_p1B),
            (my_z, "z", -1, reg_p1B),
        ]

        phase2 = []
        p2_store = []

        def store_A(cc, lo, hi):
            for y2 in range(lo, hi):
                r0 = RX + (y2 * NZ + cc) * S
                out_ref[pl.ds(r0, H), :] = (
                    abuf[pl.ds(cc, 1), pl.ds(y2, 1)].reshape(H, n).astype(f32)
                )

        def store_B(cc, lo, hi):
            for z2 in range(lo, hi):
                r0 = RX + (cc * NZ + z2) * S + H
                out_ref[pl.ds(r0, H), :] = (
                    bbuf[pl.ds(cc, 1), pl.ds(z2, 1)].reshape(H, n).astype(f32)
                )

        for lo, hi in ((0, 2), (2, 4)):
            for d in (+1, -1):
                phase2.append(
                    (my_z, "z", d,
                     lambda c, lo=lo, hi=hi: abuf.at[pl.ds(c, 1), pl.ds(lo, hi - lo)])
                )
                p2_store.append(lambda c, lo=lo, hi=hi: store_A(c, lo, hi))
            for d in (+1, -1):
                phase2.append(
                    (my_y, "y", d,
                     lambda c, lo=lo, hi=hi: bbuf.at[pl.ds(c, 1), pl.ds(lo, hi - lo)])
                )
                p2_store.append(lambda c, lo=lo, hi=hi: store_B(c, lo, hi))

        def send_cond_chunk(pos, d, s):
            if d == +1:
                return (pos < 3) & (pos - s >= 0), jnp.clip(pos - s, 0, 3)
            return (pos > 0) & (pos + s <= 3), jnp.clip(pos + s, 0, 3)

        def recv_cond_chunk(pos, d, s):
            if d == +1:
                return (pos > 0) & (pos - 1 - s >= 0), jnp.clip(pos - 1 - s, 0, 3)
            return (pos < 3) & (pos + 1 + s <= 3), jnp.clip(pos + 1 + s, 0, 3)

        def mk(reg, cc, fidx, s, axis, d):
            return pltpu.make_async_remote_copy(
                src_ref=reg(cc),
                dst_ref=reg(cc),
                send_sem=ssems.at[fidx, s],
                recv_sem=rsems.at[fidx, s],
                device_id=dev(axis, d),
                device_id_type=MESH,
            )

        def emit_sends(flows, base, s):
            for fi, (pos, axis, d, reg) in enumerate(flows):
                cond, cc = send_cond_chunk(pos, d, s)

                @pl.when(cond)
                def _(reg=reg, cc=cc, fidx=base + fi, s=s, axis=axis, d=d):
                    mk(reg, cc, fidx, s, axis, d).start()

        def emit_recv_waits(flows, base, s):
            for fi, (pos, axis, d, reg) in enumerate(flows):
                cond, cc = recv_cond_chunk(pos, d, s)

                @pl.when(cond)
                def _(reg=reg, cc=cc, fidx=base + fi, s=s, axis=axis, d=d):
                    mk(reg, cc, fidx, s, axis, -d).wait_recv()

        def emit_wait_sends(flows, base):
            for s in range(STEPS):
                for fi, (pos, axis, d, reg) in enumerate(flows):
                    cond, cc = send_cond_chunk(pos, d, s)

                    @pl.when(cond)
                    def _(reg=reg, cc=cc, fidx=base + fi, s=s, axis=axis, d=d):
                        mk(reg, cc, fidx, s, axis, d).wait_send()

        def p2_stores(s):
            for fi, (pos, axis, d, reg) in enumerate(phase2):
                cond, cc = recv_cond_chunk(pos, d, s)

                @pl.when(cond)
                def _(store=p2_store[fi], cc=cc):
                    store(cc)

        def store_own():
            for y2 in range(NY):
                r0 = RX + (y2 * NZ + my_z) * S
                out_ref[pl.ds(r0, H), :] = (
                    abuf[pl.ds(my_z, 1), pl.ds(y2, 1)].reshape(H, n).astype(f32)
                )
            for z2 in range(NZ):
                r0 = RX + (my_y * NZ + z2) * S + H
                out_ref[pl.ds(r0, H), :] = (
                    bbuf[pl.ds(my_y, 1), pl.ds(z2, 1)].reshape(H, n).astype(f32)
                )

        for s in range(STEPS):
            emit_sends(phase1, 0, s)
            emit_recv_waits(phase1, 0, s)

        emit_sends(phase2, 4, 0)
        store_own()
        emit_recv_waits(phase2, 4, 0)

        emit_sends(phase2, 4, 1)
        p2_stores(0)
        rbx1.wait()
        out_ref[0:RXH, :] = x_ref[0:RXH, :] + rbulk[0:RXH, :].astype(f32)
        emit_recv_waits(phase2, 4, 1)

        emit_sends(phase2, 4, 2)
        p2_stores(1)
        emit_recv_waits(phase2, 4, 2)

        p2_stores(2)
        rbx2.wait()
        out_ref[RXH:RX, :] = x_ref[RXH:RX, :] + rbulk[RXH:RX, :].astype(f32)

        emit_wait_sends(phase1, 0)
        emit_wait_sends(phase2, 4)

    return pl.pallas_call(
        body,
        out_shape=jax.ShapeDtypeStruct((m, n), f32),
        in_specs=[pl.BlockSpec(memory_space=pltpu.VMEM)],
        out_specs=pl.BlockSpec(memory_space=pltpu.VMEM),
        scratch_shapes=[
            pltpu.VMEM((S, n), jnp.bfloat16),
            pltpu.VMEM((S, n), jnp.bfloat16),
            pltpu.VMEM((RX, n), jnp.bfloat16),
            pltpu.VMEM((RX, n), jnp.bfloat16),
            pltpu.VMEM((NZ, NY, H, n), jnp.bfloat16),
            pltpu.VMEM((NY, NZ, H, n), jnp.bfloat16),
            pltpu.SemaphoreType.DMA,
            pltpu.SemaphoreType.DMA,
            pltpu.SemaphoreType.DMA((2,)),
            pltpu.SemaphoreType.DMA((2,)),
            pltpu.SemaphoreType.DMA((12, STEPS)),
            pltpu.SemaphoreType.DMA((12, STEPS)),
        ],
        compiler_params=pltpu.CompilerParams(collective_id=0),
    )(x)
